# Initial kernel scaffold; baseline (speedup 1.0000x reference)
import jax
import jax.numpy as jnp
from jax.experimental import pallas as pl


def kernel(x, edge_index, batch, y, Wl0, bl0, Wr0, Wl1, bl1, Wr1, gamma0, beta0, Wg, bg, pb, pm):
    raise NotImplementedError("write your pallas kernel here")


# speedup vs baseline: 2.4335x; 2.4335x over previous
import functools

import jax
import jax.numpy as jnp
from jax import lax
from jax.experimental import pallas as pl
from jax.experimental.pallas import tpu as pltpu
from jax.experimental.pallas import tpu_sc as plsc

NC = 2
NS = 16
K = 128


def _sc_agg_body(nchunk, ce, rt, with_cnt, *refs):
    if with_cnt:
        (table, src_h, dst_h, zrow_h, zcnt_h, ones_h,
         acc_o, cnt_o,
         src_v, dst_v, rows_v, ones_v, zbuf16, acc_sh, cnt_sh, sem) = refs
    else:
        (table, src_h, dst_h, zrow_h,
         acc_o,
         src_v, dst_v, rows_v, acc_sh, sem) = refs
    c = lax.axis_index("c")
    s = lax.axis_index("s")
    npad = rt * NS

    pltpu.sync_copy(zrow_h, rows_v)
    for r in range(rt // K):
        pltpu.sync_copy(rows_v, acc_sh.at[pl.ds(s * rt + r * K, K)])
    if with_cnt:
        pltpu.sync_copy(zcnt_h, zbuf16)
        for r in range(rt // K):
            pltpu.sync_copy(zbuf16, cnt_sh.at[pl.ds(s * rt + r * K, K)])
        pltpu.sync_copy(ones_h, ones_v)
    plsc.subcore_barrier()

    base0 = (c * NS + s) * ce

    def chunk(j, carry):
        base = base0 + j * K
        pltpu.sync_copy(src_h.at[pl.ds(base, K)], src_v)
        pltpu.sync_copy(dst_h.at[pl.ds(base, K)], dst_v)
        pltpu.async_copy(table.at[src_v], rows_v, sem).wait()
        pltpu.sync_copy(rows_v, acc_sh.at[dst_v], add=True)
        if with_cnt:
            pltpu.sync_copy(ones_v, cnt_sh.at[dst_v], add=True)
        return carry

    lax.fori_loop(0, nchunk, chunk, 0)
    plsc.subcore_barrier()

    for r in range(rt // K):
        pltpu.sync_copy(acc_sh.at[pl.ds(s * rt + r * K, K)], rows_v)
        pltpu.sync_copy(rows_v, acc_o.at[pl.ds(c * npad + s * rt + r * K, K)])
    if with_cnt:
        for r in range(rt // K):
            pltpu.sync_copy(cnt_sh.at[pl.ds(s * rt + r * K, K)], zbuf16)
            pltpu.sync_copy(zbuf16, cnt_o.at[pl.ds(c * npad + s * rt + r * K, K)])


def _make_sc_agg(npad, d, epad, with_cnt):
    nw = NC * NS
    ce = epad // nw
    nchunk = ce // K
    rt = npad // NS
    mesh = plsc.VectorSubcoreMesh(core_axis_name="c", subcore_axis_name="s",
                                  num_cores=NC, num_subcores=NS)
    out_type = [jax.ShapeDtypeStruct((NC * npad, d), jnp.float32)]
    scratch = [
        pltpu.VMEM((K,), jnp.int32),
        pltpu.VMEM((K,), jnp.int32),
        pltpu.VMEM((K, d), jnp.float32),
    ]
    if with_cnt:
        out_type.append(jax.ShapeDtypeStruct((NC * npad, 16), jnp.float32))
        scratch.append(pltpu.VMEM((K, 16), jnp.float32))
        scratch.append(pltpu.VMEM((K, 16), jnp.float32))
    scratch.append(pltpu.VMEM_SHARED((npad, d), jnp.float32))
    if with_cnt:
        scratch.append(pltpu.VMEM_SHARED((npad, 16), jnp.float32))
    scratch.append(pltpu.SemaphoreType.DMA)
    return pl.kernel(
        functools.partial(_sc_agg_body, nchunk, ce, rt, with_cnt),
        out_type=out_type, mesh=mesh, scratch_types=scratch,
        name="sc_edge_agg_cnt" if with_cnt else "sc_edge_agg")


def _tc_layer0_body(x, accA, accB, cntA, cntB, WlT, bl, WrT, gamma, beta, out):
    cnt = jnp.maximum(cntA[:, 0:1] + cntB[:, 0:1], 1.0)
    agg = (accA[...] + accB[...]) / cnt
    h = (jnp.dot(agg, WlT[...], preferred_element_type=jnp.float32)
         + bl[...]
         + jnp.dot(x[...], WrT[...], preferred_element_type=jnp.float32))
    h = jnp.maximum(h, 0.0)
    m = jnp.mean(h, axis=0, keepdims=True)
    v = jnp.mean((h - m) * (h - m), axis=0, keepdims=True)
    out[...] = (h - m) * lax.rsqrt(v + 1e-5) * gamma[...] + beta[...]


def _tc_layer1_body(g, h0, accA, accB, cntA, cntB, WlT, bl, WrT,
                    batch_row, batch_col, y, WgmT, WgxT, bg, pb, pm, out):
    n = h0.shape[0]
    cnt = jnp.maximum(cntA[:, 0:1] + cntB[:, 0:1], 1.0)
    agg = (accA[...] + accB[...]) / cnt
    h1 = (jnp.dot(agg, WlT[...], preferred_element_type=jnp.float32)
          + bl[...]
          + jnp.dot(h0[...], WrT[...], preferred_element_type=jnp.float32))
    h1 = jnp.maximum(h1, 0.0)

    gid = lax.broadcasted_iota(jnp.int32, (g, n), 0)
    onehot = (batch_row[...] == gid).astype(jnp.float32)
    ssum = jnp.dot(onehot, h1, preferred_element_type=jnp.float32)
    cntb = jnp.sum(onehot, axis=1, keepdims=True)
    meanp = ssum / jnp.maximum(cntb, 1.0)

    giota = lax.broadcasted_iota(jnp.int32, (g, 1), 0)
    bcol = batch_col[...]

    def mbody(gi, acc):
        vals = jnp.where(bcol == gi, h1, -1.0)
        mg = jnp.max(vals, axis=0, keepdims=True)
        return jnp.where(giota == gi, mg, acc)

    maxp = lax.fori_loop(0, g, mbody, jnp.full((g, h0.shape[1]), -1.0,
                                               dtype=jnp.float32))
    maxp = jnp.maximum(maxp, 0.0)

    rep = (jnp.dot(meanp, WgmT[...], preferred_element_type=jnp.float32)
           + jnp.dot(maxp, WgxT[...], preferred_element_type=jnp.float32)
           + bg[...])
    nrm = jnp.sqrt(jnp.sum(rep * rep, axis=1, keepdims=True))
    feat = rep / jnp.maximum(nrm, 1e-12)
    pbv = pb[...]
    pmv = pm[...]
    pbn = pbv / jnp.maximum(jnp.sqrt(jnp.sum(pbv * pbv)), 1e-12)
    pmn = pmv / jnp.maximum(jnp.sqrt(jnp.sum(pmv * pmv)), 1e-12)
    cos_b = jnp.sum(feat * pbn, axis=1, keepdims=True)
    cos_m = jnp.sum(feat * pmn, axis=1, keepdims=True)
    yv = y[...]
    cb = jnp.where(yv == 1, cos_b * cos_b, (1.0 - cos_b) * (1.0 - cos_b))
    cm = jnp.where(yv == 0, cos_m * cos_m, (1.0 - cos_m) * (1.0 - cos_m))
    out[...] = jnp.sum(cb + cm, keepdims=True)


def kernel(x, edge_index, batch, y, Wl0, bl0, Wr0, Wl1, bl1, Wr1,
           gamma0, beta0, Wg, bg, pb, pm):
    n, d = x.shape
    e = edge_index.shape[1]
    g = y.shape[0]
    hid = Wl0.shape[0]

    npad = ((n + 16 * NS - 1) // (16 * NS) + 1) * (16 * NS)
    nw = NC * NS
    epad = ((e + nw * K - 1) // (nw * K)) * (nw * K)

    src = edge_index[0]
    dst = edge_index[1]
    pad = epad - e
    if pad:
        pr = jnp.arange(pad, dtype=jnp.int32)
        src = jnp.concatenate([src, pr % n])
        dst = jnp.concatenate([dst, n + pr % (npad - n)])

    zrow = jnp.zeros((K, d), jnp.float32)
    zcnt = jnp.zeros((K, 16), jnp.float32)
    ones = jnp.ones((K, 16), jnp.float32)

    sc_agg0 = _make_sc_agg(npad, d, epad, False)
    sc_agg1 = _make_sc_agg(npad, hid, epad, False)

    (acc0,) = sc_agg0(x, src, dst, zrow)
    accA, accB = acc0[:n], acc0[npad:npad + n]
    cnt_tmp = jax.ops.segment_sum(
        jnp.ones((epad, 16), jnp.float32), dst, num_segments=npad + 256)[:n]
    cntA, cntB = cnt_tmp, jnp.zeros_like(cnt_tmp)

    h0 = pl.pallas_call(
        _tc_layer0_body,
        out_shape=jax.ShapeDtypeStruct((n, hid), jnp.float32),
        name="tc_layer0",
    )(x, accA, accB, cntA, cntB, Wl0.T, bl0.reshape(1, hid), Wr0.T,
      gamma0.reshape(1, hid), beta0.reshape(1, hid))

    (acc1,) = sc_agg1(h0, src, dst, zrow)

    loss = pl.pallas_call(
        functools.partial(_tc_layer1_body, g),
        out_shape=jax.ShapeDtypeStruct((1, 1), jnp.float32),
        name="tc_layer1_pool_loss",
    )(h0, acc1[:n], acc1[npad:npad + n], cntA, cntB, Wl1.T,
      bl1.reshape(1, hid), Wr1.T, batch.reshape(1, n), batch.reshape(n, 1),
      y.reshape(g, 1), Wg[:, :hid].T, Wg[:, hid:].T, bg.reshape(1, hid),
      pb, pm)

    return loss.reshape(())


# speedup vs baseline: 5.1399x; 2.1121x over previous
import functools

import jax
import jax.numpy as jnp
from jax import lax
from jax.experimental import pallas as pl
from jax.experimental.pallas import tpu as pltpu
from jax.experimental.pallas import tpu_sc as plsc

NC = 2
NS = 16
K = 128


def _sc_agg_body(nchunk, ce, rt, *refs):
    (table, src_h, dst_h, zrow_h, acc_o, src_v, dst_v, rows_v, acc_sh,
     sem) = refs
    c = lax.axis_index("c")
    s = lax.axis_index("s")
    npad = rt * NS

    pltpu.sync_copy(zrow_h, rows_v)
    for r in range(rt // K):
        pltpu.sync_copy(rows_v, acc_sh.at[pl.ds(s * rt + r * K, K)])
    plsc.subcore_barrier()

    base0 = (c * NS + s) * ce

    def chunk(j, carry):
        base = base0 + j * K
        pltpu.sync_copy(src_h.at[pl.ds(base, K)], src_v)
        pltpu.sync_copy(dst_h.at[pl.ds(base, K)], dst_v)
        pltpu.async_copy(table.at[src_v], rows_v, sem).wait()
        pltpu.sync_copy(rows_v, acc_sh.at[dst_v], add=True)
        return carry

    lax.fori_loop(0, nchunk, chunk, 0)
    plsc.subcore_barrier()

    for r in range(rt // K):
        pltpu.sync_copy(acc_sh.at[pl.ds(s * rt + r * K, K)], rows_v)
        pltpu.sync_copy(rows_v, acc_o.at[pl.ds(c * npad + s * rt + r * K, K)])


def _sc_cnt_body(nchunk, ce, rt, *refs):
    (dst_h, ones_h, cnt_o, dst_v, ones_v, buf_v, cnt_sh) = refs
    c = lax.axis_index("c")
    s = lax.axis_index("s")
    npad = rt * NS

    pltpu.sync_copy(ones_h.at[pl.ds(K, K)], buf_v)
    for r in range(rt // K):
        pltpu.sync_copy(buf_v, cnt_sh.at[pl.ds(s * rt + r * K, K)])
    pltpu.sync_copy(ones_h.at[pl.ds(0, K)], ones_v)
    plsc.subcore_barrier()

    base0 = (c * NS + s) * ce

    def chunk(j, carry):
        base = base0 + j * K
        pltpu.sync_copy(dst_h.at[pl.ds(base, K)], dst_v)
        pltpu.sync_copy(ones_v, cnt_sh.at[dst_v], add=True)
        return carry

    lax.fori_loop(0, nchunk, chunk, 0)
    plsc.subcore_barrier()

    for r in range(rt // K):
        pltpu.sync_copy(cnt_sh.at[pl.ds(s * rt + r * K, K)], buf_v)
        pltpu.sync_copy(buf_v, cnt_o.at[pl.ds(c * npad + s * rt + r * K, K)])


def _make_sc_cnt(npad, epad, d):
    nw = NC * NS
    ce = epad // nw
    nchunk = ce // K
    rt = npad // NS
    mesh = plsc.VectorSubcoreMesh(core_axis_name="c", subcore_axis_name="s",
                                  num_cores=NC, num_subcores=NS)
    return pl.kernel(
        functools.partial(_sc_cnt_body, nchunk, ce, rt),
        out_type=[jax.ShapeDtypeStruct((NC * npad, d), jnp.float32)],
        mesh=mesh,
        scratch_types=[
            pltpu.VMEM((K,), jnp.int32),
            pltpu.VMEM((K, d), jnp.float32),
            pltpu.VMEM((K, d), jnp.float32),
            pltpu.VMEM_SHARED((npad, d), jnp.float32),
        ],
        name="sc_edge_cnt")


def _make_sc_agg(npad, d, epad):
    nw = NC * NS
    ce = epad // nw
    nchunk = ce // K
    rt = npad // NS
    mesh = plsc.VectorSubcoreMesh(core_axis_name="c", subcore_axis_name="s",
                                  num_cores=NC, num_subcores=NS)
    return pl.kernel(
        functools.partial(_sc_agg_body, nchunk, ce, rt),
        out_type=[jax.ShapeDtypeStruct((NC * npad, d), jnp.float32)],
        mesh=mesh,
        scratch_types=[
            pltpu.VMEM((K,), jnp.int32),
            pltpu.VMEM((K,), jnp.int32),
            pltpu.VMEM((K, d), jnp.float32),
            pltpu.VMEM_SHARED((npad, d), jnp.float32),
            pltpu.SemaphoreType.DMA,
        ],
        name="sc_edge_agg")


def _tc_layer0_body(x, accA, accB, cntA, cntB, WlT, bl, WrT, gamma, beta, out):
    cnt = jnp.maximum(cntA[:, 0:1] + cntB[:, 0:1], 1.0)
    agg = (accA[...] + accB[...]) / cnt
    h = (jnp.dot(agg, WlT[...], preferred_element_type=jnp.float32)
         + bl[...]
         + jnp.dot(x[...], WrT[...], preferred_element_type=jnp.float32))
    h = jnp.maximum(h, 0.0)
    m = jnp.mean(h, axis=0, keepdims=True)
    v = jnp.mean((h - m) * (h - m), axis=0, keepdims=True)
    out[...] = (h - m) * lax.rsqrt(v + 1e-5) * gamma[...] + beta[...]


def _tc_layer1_body(g, h0, accA, accB, cntA, cntB, WlT, bl, WrT,
                    batch_row, batch_col, y, WgmT, WgxT, bg, pb, pm, out):
    n = h0.shape[0]
    cnt = jnp.maximum(cntA[:, 0:1] + cntB[:, 0:1], 1.0)
    agg = (accA[...] + accB[...]) / cnt
    h1 = (jnp.dot(agg, WlT[...], preferred_element_type=jnp.float32)
          + bl[...]
          + jnp.dot(h0[...], WrT[...], preferred_element_type=jnp.float32))
    h1 = jnp.maximum(h1, 0.0)

    gid = lax.broadcasted_iota(jnp.int32, (g, n), 0)
    onehot = (batch_row[...] == gid).astype(jnp.float32)
    ssum = jnp.dot(onehot, h1, preferred_element_type=jnp.float32)
    cntb = jnp.sum(onehot, axis=1, keepdims=True)
    meanp = ssum / jnp.maximum(cntb, 1.0)

    giota = lax.broadcasted_iota(jnp.int32, (g, 1), 0)
    bcol = batch_col[...]

    def mbody(gi, acc):
        vals = jnp.where(bcol == gi, h1, -1.0)
        mg = jnp.max(vals, axis=0, keepdims=True)
        return jnp.where(giota == gi, mg, acc)

    maxp = lax.fori_loop(0, g, mbody, jnp.full((g, h0.shape[1]), -1.0,
                                               dtype=jnp.float32))
    maxp = jnp.maximum(maxp, 0.0)

    rep = (jnp.dot(meanp, WgmT[...], preferred_element_type=jnp.float32)
           + jnp.dot(maxp, WgxT[...], preferred_element_type=jnp.float32)
           + bg[...])
    nrm = jnp.sqrt(jnp.sum(rep * rep, axis=1, keepdims=True))
    feat = rep / jnp.maximum(nrm, 1e-12)
    pbv = pb[...]
    pmv = pm[...]
    pbn = pbv / jnp.maximum(jnp.sqrt(jnp.sum(pbv * pbv)), 1e-12)
    pmn = pmv / jnp.maximum(jnp.sqrt(jnp.sum(pmv * pmv)), 1e-12)
    cos_b = jnp.sum(feat * pbn, axis=1, keepdims=True)
    cos_m = jnp.sum(feat * pmn, axis=1, keepdims=True)
    yv = y[...]
    cb = jnp.where(yv == 1, cos_b * cos_b, (1.0 - cos_b) * (1.0 - cos_b))
    cm = jnp.where(yv == 0, cos_m * cos_m, (1.0 - cos_m) * (1.0 - cos_m))
    out[...] = jnp.sum(cb + cm, keepdims=True)


def kernel(x, edge_index, batch, y, Wl0, bl0, Wr0, Wl1, bl1, Wr1,
           gamma0, beta0, Wg, bg, pb, pm):
    n, d = x.shape
    e = edge_index.shape[1]
    g = y.shape[0]
    hid = Wl0.shape[0]

    npad = -(-(n + 1) // (NS * K)) * (NS * K)
    nw = NC * NS
    epad = ((e + nw * K - 1) // (nw * K)) * (nw * K)

    src = edge_index[0]
    dst = edge_index[1]
    pad = epad - e
    if pad:
        pr = jnp.arange(pad, dtype=jnp.int32)
        src = jnp.concatenate([src, pr % n])
        dst = jnp.concatenate([dst, n + pr % (npad - n)])

    zrow = jnp.zeros((K, hid), jnp.float32)
    ones_zeros = jnp.concatenate([jnp.ones((K, hid), jnp.float32),
                                  jnp.zeros((K, hid), jnp.float32)], axis=0)

    (acc0,) = _make_sc_agg(npad, d, epad)(x, src, dst, zrow)
    accA, accB = acc0[:n], acc0[npad:npad + n]
    (cnt0,) = _make_sc_cnt(npad, epad, hid)(dst, ones_zeros)
    cntA, cntB = cnt0[:n, :16], cnt0[npad:npad + n, :16]

    h0 = pl.pallas_call(
        _tc_layer0_body,
        out_shape=jax.ShapeDtypeStruct((n, hid), jnp.float32),
        name="tc_layer0",
    )(x, accA, accB, cntA, cntB, Wl0.T, bl0.reshape(1, hid), Wr0.T,
      gamma0.reshape(1, hid), beta0.reshape(1, hid))

    (acc1,) = _make_sc_agg(npad, hid, epad)(h0, src, dst, zrow)

    loss = pl.pallas_call(
        functools.partial(_tc_layer1_body, g),
        out_shape=jax.ShapeDtypeStruct((1, 1), jnp.float32),
        name="tc_layer1_pool_loss",
    )(h0, acc1[:n], acc1[npad:npad + n], cntA, cntB, Wl1.T,
      bl1.reshape(1, hid), Wr1.T, batch.reshape(1, n), batch.reshape(n, 1),
      y.reshape(g, 1), Wg[:, :hid].T, Wg[:, hid:].T, bg.reshape(1, hid),
      pb, pm)

    return loss.reshape(())


# speedup vs baseline: 5.3820x; 1.0471x over previous
import functools

import jax
import jax.numpy as jnp
from jax import lax
from jax.experimental import pallas as pl
from jax.experimental.pallas import tpu as pltpu
from jax.experimental.pallas import tpu_sc as plsc

NC = 2
NS = 16
K = 128


def _sc_agg_body(nchunk, ce, rt, *refs):
    (table, src_h, dst_h, zrow_h, acc_o, src_v, dst_v, rows_v, acc_sh,
     sem) = refs
    c = lax.axis_index("c")
    s = lax.axis_index("s")
    npad = rt * NS

    pltpu.sync_copy(zrow_h, rows_v)
    for r in range(rt // K):
        pltpu.sync_copy(rows_v, acc_sh.at[pl.ds(s * rt + r * K, K)])
    plsc.subcore_barrier()

    base0 = (c * NS + s) * ce

    def chunk(j, carry):
        base = base0 + j * K
        pltpu.sync_copy(src_h.at[pl.ds(base, K)], src_v)
        pltpu.sync_copy(dst_h.at[pl.ds(base, K)], dst_v)
        pltpu.async_copy(table.at[src_v], rows_v, sem).wait()
        pltpu.sync_copy(rows_v, acc_sh.at[dst_v], add=True)
        return carry

    lax.fori_loop(0, nchunk, chunk, 0)
    plsc.subcore_barrier()

    for r in range(rt // K):
        pltpu.sync_copy(acc_sh.at[pl.ds(s * rt + r * K, K)], rows_v)
        pltpu.sync_copy(rows_v, acc_o.at[pl.ds(c * npad + s * rt + r * K, K)])


def _sc_cnt_body(nchunk, ce, rt, *refs):
    (dst_h, ones_h, cnt_o, dst_v, ones_v, buf_v, cnt_sh) = refs
    c = lax.axis_index("c")
    s = lax.axis_index("s")
    npad = rt * NS

    pltpu.sync_copy(ones_h.at[pl.ds(K, K)], buf_v)
    for r in range(rt // K):
        pltpu.sync_copy(buf_v, cnt_sh.at[pl.ds(s * rt + r * K, K)])
    pltpu.sync_copy(ones_h.at[pl.ds(0, K)], ones_v)
    plsc.subcore_barrier()

    base0 = (c * NS + s) * ce

    def chunk(j, carry):
        base = base0 + j * K
        pltpu.sync_copy(dst_h.at[pl.ds(base, K)], dst_v)
        pltpu.sync_copy(ones_v, cnt_sh.at[dst_v], add=True)
        return carry

    lax.fori_loop(0, nchunk, chunk, 0)
    plsc.subcore_barrier()

    for r in range(rt // K):
        pltpu.sync_copy(cnt_sh.at[pl.ds(s * rt + r * K, K)], buf_v)
        pltpu.sync_copy(buf_v, cnt_o.at[pl.ds(c * npad + s * rt + r * K, K)])


def _make_sc_cnt(npad, epad, d):
    nw = NC * NS
    ce = epad // nw
    nchunk = ce // K
    rt = npad // NS
    mesh = plsc.VectorSubcoreMesh(core_axis_name="c", subcore_axis_name="s",
                                  num_cores=NC, num_subcores=NS)
    return pl.kernel(
        functools.partial(_sc_cnt_body, nchunk, ce, rt),
        out_type=[jax.ShapeDtypeStruct((NC * npad, d), jnp.float32)],
        mesh=mesh,
        scratch_types=[
            pltpu.VMEM((K,), jnp.int32),
            pltpu.VMEM((K, d), jnp.float32),
            pltpu.VMEM((K, d), jnp.float32),
            pltpu.VMEM_SHARED((npad, d), jnp.float32),
        ],
        name="sc_edge_cnt")


def _make_sc_agg(npad, d, epad):
    nw = NC * NS
    ce = epad // nw
    nchunk = ce // K
    rt = npad // NS
    mesh = plsc.VectorSubcoreMesh(core_axis_name="c", subcore_axis_name="s",
                                  num_cores=NC, num_subcores=NS)
    return pl.kernel(
        functools.partial(_sc_agg_body, nchunk, ce, rt),
        out_type=[jax.ShapeDtypeStruct((NC * npad, d), jnp.float32)],
        mesh=mesh,
        scratch_types=[
            pltpu.VMEM((K,), jnp.int32),
            pltpu.VMEM((K,), jnp.int32),
            pltpu.VMEM((K, d), jnp.float32),
            pltpu.VMEM_SHARED((npad, d), jnp.float32),
            pltpu.SemaphoreType.DMA,
        ],
        name="sc_edge_agg")


def _tc_layer0_body(x, accA, accB, cntA, cntB, WlT, bl, WrT, gamma, beta, out):
    cnt = jnp.maximum(cntA[:, 0:1] + cntB[:, 0:1], 1.0)
    agg = (accA[...] + accB[...]) / cnt
    h = (jnp.dot(agg, WlT[...], preferred_element_type=jnp.float32)
         + bl[...]
         + jnp.dot(x[...], WrT[...], preferred_element_type=jnp.float32))
    h = jnp.maximum(h, 0.0)
    m = jnp.mean(h, axis=0, keepdims=True)
    v = jnp.mean((h - m) * (h - m), axis=0, keepdims=True)
    out[...] = (h - m) * lax.rsqrt(v + 1e-5) * gamma[...] + beta[...]


def _tc_layer1_body(g, h0, accA, accB, cntA, cntB, WlT, bl, WrT,
                    batch_row, batch_col, y, WgmT, WgxT, bg, pb, pm, out):
    n = h0.shape[0]
    cnt = jnp.maximum(cntA[:, 0:1] + cntB[:, 0:1], 1.0)
    agg = (accA[...] + accB[...]) / cnt
    h1 = (jnp.dot(agg, WlT[...], preferred_element_type=jnp.float32)
          + bl[...]
          + jnp.dot(h0[...], WrT[...], preferred_element_type=jnp.float32))
    h1 = jnp.maximum(h1, 0.0)

    gid = lax.broadcasted_iota(jnp.int32, (g, n), 0)
    onehot = (batch_row[...] == gid).astype(jnp.float32)
    ssum = jnp.dot(onehot, h1, preferred_element_type=jnp.float32)
    cntb = jnp.sum(onehot, axis=1, keepdims=True)
    meanp = ssum / jnp.maximum(cntb, 1.0)

    giota = lax.broadcasted_iota(jnp.int32, (g, 1), 0)
    bcol = batch_col[...]

    def mbody(gi, acc):
        vals = jnp.where(bcol == gi, h1, -1.0)
        mg = jnp.max(vals, axis=0, keepdims=True)
        return jnp.where(giota == gi, mg, acc)

    maxp = lax.fori_loop(0, g, mbody, jnp.full((g, h0.shape[1]), -1.0,
                                               dtype=jnp.float32))
    maxp = jnp.maximum(maxp, 0.0)

    rep = (jnp.dot(meanp, WgmT[...], preferred_element_type=jnp.float32)
           + jnp.dot(maxp, WgxT[...], preferred_element_type=jnp.float32)
           + bg[...])
    nrm = jnp.sqrt(jnp.sum(rep * rep, axis=1, keepdims=True))
    feat = rep / jnp.maximum(nrm, 1e-12)
    pbv = pb[...]
    pmv = pm[...]
    pbn = pbv / jnp.maximum(jnp.sqrt(jnp.sum(pbv * pbv)), 1e-12)
    pmn = pmv / jnp.maximum(jnp.sqrt(jnp.sum(pmv * pmv)), 1e-12)
    cos_b = jnp.sum(feat * pbn, axis=1, keepdims=True)
    cos_m = jnp.sum(feat * pmn, axis=1, keepdims=True)
    yv = y[...]
    cb = jnp.where(yv == 1, cos_b * cos_b, (1.0 - cos_b) * (1.0 - cos_b))
    cm = jnp.where(yv == 0, cos_m * cos_m, (1.0 - cos_m) * (1.0 - cos_m))
    out[...] = jnp.sum(cb + cm, keepdims=True)


def kernel(x, edge_index, batch, y, Wl0, bl0, Wr0, Wl1, bl1, Wr1,
           gamma0, beta0, Wg, bg, pb, pm):
    n, d = x.shape
    e = edge_index.shape[1]
    g = y.shape[0]
    hid = Wl0.shape[0]

    npad = -(-(n + 1) // (NS * K)) * (NS * K)
    nw = NC * NS
    epad = ((e + nw * K - 1) // (nw * K)) * (nw * K)

    src = edge_index[0]
    dst = edge_index[1]
    pad = epad - e
    if pad:
        pr = jnp.arange(pad, dtype=jnp.int32)
        src = jnp.concatenate([src, pr % n])
        dst = jnp.concatenate([dst, n + pr % (npad - n)])

    zrow = jnp.zeros((K, hid), jnp.float32)
    ones_zeros = jnp.concatenate([jnp.ones((K, hid), jnp.float32),
                                  jnp.zeros((K, hid), jnp.float32)], axis=0)

    (acc0,) = _make_sc_agg(npad, d, epad)(x, src, dst, zrow)
    accA, accB = acc0[:n], acc0[npad:npad + n]
    (cnt0,) = _make_sc_cnt(npad, epad, 32)(dst, ones_zeros[:, :32])
    cntA, cntB = cnt0[:n, :16], cnt0[npad:npad + n, :16]

    h0 = pl.pallas_call(
        _tc_layer0_body,
        out_shape=jax.ShapeDtypeStruct((n, hid), jnp.float32),
        name="tc_layer0",
    )(x, accA, accB, cntA, cntB, Wl0.T, bl0.reshape(1, hid), Wr0.T,
      gamma0.reshape(1, hid), beta0.reshape(1, hid))

    (acc1,) = _make_sc_agg(npad, hid, epad)(h0, src, dst, zrow)

    loss = pl.pallas_call(
        functools.partial(_tc_layer1_body, g),
        out_shape=jax.ShapeDtypeStruct((1, 1), jnp.float32),
        name="tc_layer1_pool_loss",
    )(h0, acc1[:n], acc1[npad:npad + n], cntA, cntB, Wl1.T,
      bl1.reshape(1, hid), Wr1.T, batch.reshape(1, n), batch.reshape(n, 1),
      y.reshape(g, 1), Wg[:, :hid].T, Wg[:, hid:].T, bg.reshape(1, hid),
      pb, pm)

    return loss.reshape(())
